# Initial kernel scaffold; baseline (speedup 1.0000x reference)
import jax
import jax.numpy as jnp
from jax.experimental import pallas as pl


def kernel(logits, label):
    raise NotImplementedError("write your pallas kernel here")


# speedup vs baseline: 2.3752x; 2.3752x over previous
import jax
import jax.numpy as jnp
from jax.experimental import pallas as pl
from jax.experimental.pallas import tpu as pltpu

_C = 1000
_SMOOTH = 0.1
_ON = 1.0 - _SMOOTH
_OFF = _SMOOTH / (_C - 1)
_ROWS_PER_BLOCK = 512


def _ce_body(x_ref, lbl_ref, out_ref, acc_ref):
    i = pl.program_id(0)
    x = x_ref[...]
    lbl = lbl_ref[...]
    r = x.shape[0]
    m = jnp.max(x, axis=1, keepdims=True)
    s = jnp.sum(jnp.exp(x - m), axis=1, keepdims=True)
    lse_sum = jnp.sum(m + jnp.log(s))
    total_sum = jnp.sum(x)
    cols = jax.lax.broadcasted_iota(jnp.int32, (r, _C), 1)
    g_sum = jnp.sum(jnp.where(cols == lbl, x, 0.0))
    c = lse_sum - _OFF * total_sum - (_ON - _OFF) * g_sum

    @pl.when(i == 0)
    def _init():
        acc_ref[0] = 0.0

    acc_ref[0] += c

    @pl.when(i == pl.num_programs(0) - 1)
    def _fin():
        out_ref[0] = acc_ref[0]


def kernel(logits, label):
    n, c = logits.shape
    r = _ROWS_PER_BLOCK
    nb = n // r
    lbl2 = label.astype(jnp.int32).reshape(n, 1)
    out = pl.pallas_call(
        _ce_body,
        grid=(nb,),
        in_specs=[
            pl.BlockSpec((r, c), lambda i: (i, 0)),
            pl.BlockSpec((r, 1), lambda i: (i, 0)),
        ],
        out_specs=pl.BlockSpec(memory_space=pltpu.SMEM),
        out_shape=jax.ShapeDtypeStruct((1,), jnp.float32),
        scratch_shapes=[pltpu.SMEM((1,), jnp.float32)],
    )(logits, lbl2)
    return out[0] * (1.0 / n)
